# baseline (speedup 1.0000x reference)
import jax
import jax.numpy as jnp
from jax import lax
from jax.experimental import pallas as pl
from jax.experimental.pallas import tpu as pltpu
from jax.experimental.pallas import tpu_sc as plsc

N = 10000
E = 320000
D = 128
D_OUT = 64
G = 64
EPS = 1e-5

NC = 2
NS = 16
NW = NC * NS
EPT = E // NS
CH = 32
GS = 16
NG = 40
NGT = NG + 2
NCHT = NGT * GS
HRNG = N // NC
ACC_H = 5120
DUMMY = 5100
RPT = ACC_H // NS
ZR = CH
DGH = NCHT // 2
DGG = 4
BLK = 1000
HIGH = lax.Precision.HIGHEST


N_HIST = 10048


def _sc_degree_body(dst_hbm, out_hbm, dstc, hist, sem):
    c = lax.axis_index("c")
    s = lax.axis_index("s")
    wid = c * NS + s
    row = wid // 2
    half = wid % 2

    zeros16 = jnp.zeros((16,), jnp.float32)

    def zero_body(i, carry):
        hist[pl.ds(i * 16, 16)] = zeros16
        return carry

    lax.fori_loop(0, N_HIST // 16, zero_body, 0)

    ones16 = jnp.ones((16,), jnp.float32)

    def hist_body(g, carry):
        pltpu.async_copy(
            dst_hbm.at[row, pl.ds(half * DGH + g * DGG, DGG)], dstc, sem
        ).wait()
        for r in range(DGG):
            for i in range(CH // 16):
                idx = dstc[r, pl.ds(i * 16, 16)]
                plsc.addupdate_scatter(hist, [idx], ones16)
        return carry

    lax.fori_loop(0, DGH // DGG, hist_body, 0)
    pltpu.sync_copy(hist, out_hbm.at[wid])


def _sc_degree(dst_w):
    return pl.kernel(
        _sc_degree_body,
        out_type=jax.ShapeDtypeStruct((NW, N_HIST), jnp.float32),
        mesh=plsc.VectorSubcoreMesh(core_axis_name="c", subcore_axis_name="s"),
        scratch_types=[
            pltpu.VMEM((DGG, CH), jnp.int32),
            pltpu.VMEM((N_HIST,), jnp.float32),
            pltpu.SemaphoreType.DMA,
        ],
        compiler_params=pltpu.CompilerParams(needs_layout_passes=False),
    )(dst_w)


def _sc_scatter_body(hd_hbm, src_hbm, dst_hbm, out_hbm,
                     srcg, dstg, rows0, rows1, acc,
                     semg0, semg1, sem0, sem1, sems0, sems1):
    c = lax.axis_index("c")
    s = lax.axis_index("s")
    base = c * HRNG

    zeros16 = jnp.zeros((16,), jnp.float32)

    def zb_body(i, carry):
        for k in range(D // 16):
            rows0[i, pl.ds(k * 16, 16)] = zeros16
        return carry

    lax.fori_loop(0, ZR, zb_body, 0)

    def zacc_body(t, carry):
        pltpu.sync_copy(rows0, acc.at[pl.ds(s * RPT + t * ZR, ZR)])
        return carry

    lax.fori_loop(0, RPT // ZR, zacc_body, 0)
    plsc.subcore_barrier()

    def idesc(g, slot, sem):
        return (
            pltpu.make_async_copy(src_hbm.at[s, pl.ds(g * GS, GS)],
                                  srcg.at[slot], sem),
            pltpu.make_async_copy(dst_hbm.at[s, pl.ds(g * GS, GS)],
                                  dstg.at[slot], sem),
        )

    def istart(g, slot, sem):
        a, b = idesc(g, slot, sem)
        a.start()
        b.start()

    def iwait(g, slot, sem):
        a, b = idesc(g, slot, sem)
        a.wait()
        b.wait()

    def remap(slot):
        def rm_body(r, carry):
            for i in range(CH // 16):
                v = dstg[slot, r, pl.ds(i * 16, 16)]
                t = v - base
                ok = (t >= 0) & (t < HRNG)
                dstg[slot, r, pl.ds(i * 16, 16)] = jnp.where(ok, t, DUMMY)
            return carry

        lax.fori_loop(0, GS, rm_body, 0)

    def gdesc(j, rows, sem):
        slot = (j // GS) % 2
        return pltpu.make_async_copy(hd_hbm.at[srcg.at[slot, j % GS]],
                                     rows, sem)

    istart(0, 0, semg0)
    iwait(0, 0, semg0)
    remap(0)
    istart(1, 1, semg1)
    gdesc(0, rows0, sem0).start()
    gdesc(1, rows1, sem1).start()

    def make_inner(base):
        def inner(jj, carry2):
            j0 = base + 2 * jj
            j1 = j0 + 1
            gdesc(j0, rows0, sem0).wait()
            s0 = pltpu.async_copy(
                rows0, acc.at[dstg.at[(j0 // GS) % 2, j0 % GS]], sems0,
                add=True)
            gdesc(j1, rows1, sem1).wait()
            s1 = pltpu.async_copy(
                rows1, acc.at[dstg.at[(j1 // GS) % 2, j1 % GS]], sems1,
                add=True)
            s0.wait()
            gdesc(j0 + 2, rows0, sem0).start()
            s1.wait()
            gdesc(j1 + 2, rows1, sem1).start()
            return carry2

        return inner

    def outer2(gg, carry):
        g0 = 2 * gg
        a, b = idesc(g0 + 1, 1, semg1)
        a.wait()
        b.wait()
        remap(1)
        lax.fori_loop(0, GS // 2, make_inner(g0 * GS), 0)
        istart(g0 + 2, 0, semg0)

        a, b = idesc(g0 + 2, 0, semg0)
        a.wait()
        b.wait()
        remap(0)
        lax.fori_loop(0, GS // 2, make_inner((g0 + 1) * GS), 0)
        istart(g0 + 3, 1, semg1)
        return carry

    lax.fori_loop(0, NG // 2, outer2, 0)

    a, b = idesc(NG + 1, 1, semg1)
    a.wait()
    b.wait()
    gdesc(NG * GS, rows0, sem0).wait()
    gdesc(NG * GS + 1, rows1, sem1).wait()

    plsc.subcore_barrier()
    pltpu.sync_copy(acc.at[pl.ds(s * RPT, RPT)],
                    out_hbm.at[c, pl.ds(s * RPT, RPT)])


def _sc_scatter(hd, src_w, dst_w):
    return pl.kernel(
        _sc_scatter_body,
        out_type=jax.ShapeDtypeStruct((NC, ACC_H, D), jnp.float32),
        mesh=plsc.VectorSubcoreMesh(core_axis_name="c", subcore_axis_name="s"),
        scratch_types=[
            pltpu.VMEM((2, GS, CH), jnp.int32),
            pltpu.VMEM((2, GS, CH), jnp.int32),
            pltpu.VMEM((CH, D), jnp.float32),
            pltpu.VMEM((CH, D), jnp.float32),
            pltpu.VMEM_SHARED((ACC_H, D), jnp.float32),
            pltpu.SemaphoreType.DMA,
            pltpu.SemaphoreType.DMA,
            pltpu.SemaphoreType.DMA,
            pltpu.SemaphoreType.DMA,
            pltpu.SemaphoreType.DMA,
            pltpu.SemaphoreType.DMA,
        ],
    )(hd, src_w, dst_w)


def _tc_dis_body(hists_ref, dis_ref):
    deg = jnp.sum(hists_ref[...], axis=0) + 1.0
    dis_ref[...] = lax.rsqrt(deg)[:, None]


def _tc_dis(hists):
    return pl.pallas_call(
        _tc_dis_body,
        out_shape=jax.ShapeDtypeStruct((N_HIST, 1), jnp.float32),
    )(hists)


def _tc_hd1_body(x_ref, w1_ref, dis_ref, hd1_ref):
    h = jnp.dot(x_ref[...], w1_ref[...], preferred_element_type=jnp.float32,
                precision=HIGH)
    hd1_ref[...] = h * dis_ref[...]


def _tc_hd1(x, w1, dis):
    return pl.pallas_call(
        _tc_hd1_body,
        grid=(N // BLK,),
        in_specs=[
            pl.BlockSpec((BLK, D), lambda i: (i, 0)),
            pl.BlockSpec((D, D), lambda i: (0, 0)),
            pl.BlockSpec((BLK, 1), lambda i: (i, 0)),
        ],
        out_specs=pl.BlockSpec((BLK, D), lambda i: (i, 0)),
        out_shape=jax.ShapeDtypeStruct((N, D), jnp.float32),
    )(x, w1, dis)


def _acc_spec():
    return pl.BlockSpec((1, BLK, D), lambda i: (i // 5, i % 5, 0))


def _tc_layer_body(acc_ref, hd_ref, dis_ref, b_ref, g_ref, be_ref, w_ref,
                   h_ref, hdn_ref):
    dis = dis_ref[...]
    agg = acc_ref[0] + hd_ref[...]
    out = dis * agg + b_ref[0, :]
    bnscale = g_ref[0, :] / jnp.sqrt(1.0 + EPS)
    h = jnp.maximum(out * bnscale + be_ref[0, :], 0.0)
    h_ref[...] = h
    hdn = jnp.dot(h, w_ref[...], preferred_element_type=jnp.float32,
                  precision=HIGH)
    hdn_ref[...] = hdn * dis


def _tc_layer(acc, hd, dis, b, g, be, w):
    return pl.pallas_call(
        _tc_layer_body,
        grid=(N // BLK,),
        in_specs=[
            _acc_spec(),
            pl.BlockSpec((BLK, D), lambda i: (i, 0)),
            pl.BlockSpec((BLK, 1), lambda i: (i, 0)),
            pl.BlockSpec((1, D), lambda i: (0, 0)),
            pl.BlockSpec((1, D), lambda i: (0, 0)),
            pl.BlockSpec((1, D), lambda i: (0, 0)),
            pl.BlockSpec((D, D), lambda i: (0, 0)),
        ],
        out_specs=[
            pl.BlockSpec((BLK, D), lambda i: (i, 0)),
            pl.BlockSpec((BLK, D), lambda i: (i, 0)),
        ],
        out_shape=[
            jax.ShapeDtypeStruct((N, D), jnp.float32),
            jax.ShapeDtypeStruct((N, D), jnp.float32),
        ],
    )(acc, hd, dis, b, g, be, w)


def _tc_final_body(acc_ref, hd_ref, dis_ref, b_ref, g_ref, be_ref,
                   x_ref, h1_ref, batch_ref,
                   p0_ref, p1_ref, p2_ref, pb0_ref, pb1_ref, pb2_ref,
                   score_ref):
    agg = acc_ref[0] + hd_ref[...]
    out = dis_ref[...] * agg + b_ref[0, :]
    bnscale = g_ref[0, :] / jnp.sqrt(1.0 + EPS)
    h2 = jnp.maximum(out * bnscale + be_ref[0, :], 0.0)

    t = jnp.dot(x_ref[...], p0_ref[...], preferred_element_type=jnp.float32,
                precision=HIGH)
    t += jnp.dot(h1_ref[...], p1_ref[...], preferred_element_type=jnp.float32,
                 precision=HIGH)
    t += jnp.dot(h2, p2_ref[...], preferred_element_type=jnp.float32,
                 precision=HIGH)

    b = batch_ref[0, 0, :]
    gio = lax.broadcasted_iota(jnp.int32, (BLK, G), 1)
    onehot = (b[:, None] == gio).astype(jnp.float32)
    contrib = lax.dot_general(onehot, t, (((0,), (0,)), ((), ())),
                              preferred_element_type=jnp.float32,
                              precision=HIGH)

    @pl.when(pl.program_id(0) == 0)
    def _():
        pbs = pb0_ref[0, :] + pb1_ref[0, :] + pb2_ref[0, :]
        score_ref[...] = jnp.broadcast_to(pbs[None, :], (G, D_OUT))

    score_ref[...] += contrib


def _tc_final(acc, hd, dis, b, g, be, x, h1, batch_r, p0, p1, p2,
              pb0, pb1, pb2):
    return pl.pallas_call(
        _tc_final_body,
        grid=(N // BLK,),
        in_specs=[
            _acc_spec(),
            pl.BlockSpec((BLK, D), lambda i: (i, 0)),
            pl.BlockSpec((BLK, 1), lambda i: (i, 0)),
            pl.BlockSpec((1, D), lambda i: (0, 0)),
            pl.BlockSpec((1, D), lambda i: (0, 0)),
            pl.BlockSpec((1, D), lambda i: (0, 0)),
            pl.BlockSpec((BLK, D), lambda i: (i, 0)),
            pl.BlockSpec((BLK, D), lambda i: (i, 0)),
            pl.BlockSpec((1, 1, BLK), lambda i: (i, 0, 0)),
            pl.BlockSpec((D, D_OUT), lambda i: (0, 0)),
            pl.BlockSpec((D, D_OUT), lambda i: (0, 0)),
            pl.BlockSpec((D, D_OUT), lambda i: (0, 0)),
            pl.BlockSpec((1, D_OUT), lambda i: (0, 0)),
            pl.BlockSpec((1, D_OUT), lambda i: (0, 0)),
            pl.BlockSpec((1, D_OUT), lambda i: (0, 0)),
        ],
        out_specs=pl.BlockSpec((G, D_OUT), lambda i: (0, 0)),
        out_shape=jax.ShapeDtypeStruct((G, D_OUT), jnp.float32),
    )(acc, hd, dis, b, g, be, x, h1, batch_r, p0, p1, p2, pb0, pb1, pb2)


def kernel(x, edge_index, batch, W1, b1, g1, be1, W2, b2, g2, be2,
           P0, pb0, P1, pb1, P2, pb2):
    src, dst = edge_index[0], edge_index[1]
    pad = NCHT * CH - EPT
    src_w = jnp.pad(src.reshape(NS, EPT), ((0, 0), (0, pad))).reshape(NS, NCHT, CH)
    dst_w = jnp.pad(dst.reshape(NS, EPT), ((0, 0), (0, pad)),
                    constant_values=N).reshape(NS, NCHT, CH)

    hists = _sc_degree(dst_w)
    dis = _tc_dis(hists)

    hd1 = _tc_hd1(x, W1, dis)
    acc1 = _sc_scatter(hd1, src_w, dst_w)
    h1, hd2 = _tc_layer(acc1, hd1, dis,
                        b1.reshape(1, D), g1.reshape(1, D), be1.reshape(1, D),
                        W2)
    acc2 = _sc_scatter(hd2, src_w, dst_w)

    batch_r = batch.reshape(N // BLK, 1, BLK)
    score = _tc_final(acc2, hd2, dis,
                      b2.reshape(1, D), g2.reshape(1, D), be2.reshape(1, D),
                      x, h1, batch_r,
                      P0, P1, P2,
                      pb0.reshape(1, D_OUT), pb1.reshape(1, D_OUT),
                      pb2.reshape(1, D_OUT))
    return score


# speedup vs baseline: 1.0980x; 1.0980x over previous
import jax
import jax.numpy as jnp
from jax import lax
from jax.experimental import pallas as pl
from jax.experimental.pallas import tpu as pltpu
from jax.experimental.pallas import tpu_sc as plsc

N = 10000
E = 320000
D = 128
D_OUT = 64
G = 64
EPS = 1e-5

NC = 2
NS = 16
NW = NC * NS
EPT = E // NS
CH = 32
GS = 16
NG = 42
NGT = NG + 2
NCHT = NGT * GS
CAPC = 704
HRNG = N // NC
ACC_H = 5120
DUMMY = 5100
RPT = ACC_H // NS
ZR = CH
DGH = NCHT // 2
DGG = 4
BLK = 1000
HIGH = lax.Precision.HIGHEST


N_HIST = 10048


def _sc_degree_body(dst_hbm, out_hbm, dstc, hist, sem):
    c = lax.axis_index("c")
    s = lax.axis_index("s")
    wid = c * NS + s
    row = wid // 2
    half = wid % 2

    zeros16 = jnp.zeros((16,), jnp.float32)

    def zero_body(i, carry):
        hist[pl.ds(i * 16, 16)] = zeros16
        return carry

    lax.fori_loop(0, N_HIST // 16, zero_body, 0)

    ones16 = jnp.ones((16,), jnp.float32)

    def hist_body(g, carry):
        pltpu.async_copy(
            dst_hbm.at[row, pl.ds(half * DGH + g * DGG, DGG)], dstc, sem
        ).wait()
        for r in range(DGG):
            for i in range(CH // 16):
                idx = dstc[r, pl.ds(i * 16, 16)]
                plsc.addupdate_scatter(hist, [idx], ones16)
        return carry

    lax.fori_loop(0, DGH // DGG, hist_body, 0)
    pltpu.sync_copy(hist, out_hbm.at[wid])


def _sc_degree(dst_w):
    return pl.kernel(
        _sc_degree_body,
        out_type=jax.ShapeDtypeStruct((NW, N_HIST), jnp.float32),
        mesh=plsc.VectorSubcoreMesh(core_axis_name="c", subcore_axis_name="s"),
        scratch_types=[
            pltpu.VMEM((DGG, CH), jnp.int32),
            pltpu.VMEM((N_HIST,), jnp.float32),
            pltpu.SemaphoreType.DMA,
        ],
        compiler_params=pltpu.CompilerParams(needs_layout_passes=False),
    )(dst_w)


STG = 576


def _sc_scatter_body(hd_hbm, src_hbm, dst_hbm, out_hbm, lsrc_hbm, ldst_hbm,
                     srcg, dstg, ibufS, ibufD, rows0, rows1, cS, cD, acc,
                     semg0, semg1, sem0, sem1, semf0, semf1):
    c = lax.axis_index("c")
    s = lax.axis_index("s")
    base = c * HRNG
    tb = (c * NS + s) * CAPC * CH

    zeros16 = jnp.zeros((16,), jnp.float32)
    zeros16i = jnp.zeros((16,), jnp.int32)
    dummy16 = jnp.full((16,), DUMMY, jnp.int32)

    def zb_body(i, carry):
        for k in range(D // 16):
            rows0[i, pl.ds(k * 16, 16)] = zeros16
        return carry

    lax.fori_loop(0, ZR, zb_body, 0)

    def zacc_body(t, carry):
        pltpu.sync_copy(rows0, acc.at[pl.ds(s * RPT + t * ZR, ZR)])
        return carry

    lax.fori_loop(0, RPT // ZR, zacc_body, 0)

    def idesc(g, slot, sem):
        return (
            pltpu.make_async_copy(src_hbm.at[s, pl.ds(g * GS, GS)],
                                  srcg.at[slot], sem),
            pltpu.make_async_copy(dst_hbm.at[s, pl.ds(g * GS, GS)],
                                  dstg.at[slot], sem),
        )

    def istart(g, slot, sem):
        a, b = idesc(g, slot, sem)
        a.start()
        b.start()

    def iwait(g, slot, sem):
        a, b = idesc(g, slot, sem)
        a.wait()
        b.wait()

    def fdesc(slot, k, pos, semf):
        return (
            pltpu.make_async_copy(cS.at[pl.ds(slot * STG + k * CH, CH)],
                                  lsrc_hbm.at[pl.ds(tb + (pos + k) * CH, CH)],
                                  semf),
            pltpu.make_async_copy(cD.at[pl.ds(slot * STG + k * CH, CH)],
                                  ldst_hbm.at[pl.ds(tb + (pos + k) * CH, CH)],
                                  semf),
        )

    def pa_process(slot, pos, semf, nf_prev):
        def drain_body(k, cc):
            a, b = fdesc(slot, 0, 0, semf)
            a.wait()
            b.wait()
            return cc

        lax.fori_loop(0, nf_prev, drain_body, 0)

        def cg(rr, off):
            for i in range(CH // 16):
                vs = srcg[slot, rr, pl.ds(i * 16, 16)]
                vd = dstg[slot, rr, pl.ds(i * 16, 16)]
                t = vd - base
                m = (t >= 0) & (t < HRNG)
                cnt = plsc.all_reduce_population_count(m)[0]
                plsc.store_compressed(cS.at[pl.ds(slot * STG + off, 16)],
                                      vs, mask=m)
                plsc.store_compressed(cD.at[pl.ds(slot * STG + off, 16)],
                                      t, mask=m)
                off = off + cnt
            return off

        off = lax.fori_loop(0, GS, cg, 0)

        cS[pl.ds(slot * STG + off, 16)] = zeros16i
        cS[pl.ds(slot * STG + off + 16, 16)] = zeros16i
        cD[pl.ds(slot * STG + off, 16)] = dummy16
        cD[pl.ds(slot * STG + off + 16, 16)] = dummy16
        nf = (off + CH - 1) // CH

        def fl(k, cc):
            a, b = fdesc(slot, k, pos, semf)
            a.start()
            b.start()
            return cc

        lax.fori_loop(0, nf, fl, 0)
        return pos + nf, nf

    istart(0, 0, semg0)
    iwait(0, 0, semg0)
    istart(1, 1, semg1)

    def pa_outer(gg, carry):
        pos, nfA, nfB = carry
        g0 = 2 * gg
        pos, nfA = pa_process(0, pos, semf0, nfA)
        istart(g0 + 2, 0, semg0)
        iwait(g0 + 1, 1, semg1)
        pos, nfB = pa_process(1, pos, semf1, nfB)
        istart(g0 + 3, 1, semg1)
        iwait(g0 + 2, 0, semg0)
        return pos, nfA, nfB

    pos, nfA, nfB = lax.fori_loop(0, NG // 2, pa_outer, (0, 0, 0))
    iwait(NG + 1, 1, semg1)

    def drainA(k, cc):
        a, b = fdesc(0, 0, 0, semf0)
        a.wait()
        b.wait()
        return cc

    def drainB(k, cc):
        a, b = fdesc(1, 0, 0, semf1)
        a.wait()
        b.wait()
        return cc

    lax.fori_loop(0, nfA, drainA, 0)
    lax.fori_loop(0, nfB, drainB, 0)

    for k in range(2 * CH // 16):
        cS[pl.ds(k * 16, 16)] = zeros16i
        cD[pl.ds(k * 16, 16)] = dummy16
    pltpu.sync_copy(cS.at[pl.ds(0, 2 * CH)],
                    lsrc_hbm.at[pl.ds(tb + pos * CH, 2 * CH)])
    pltpu.sync_copy(cD.at[pl.ds(0, 2 * CH)],
                    ldst_hbm.at[pl.ds(tb + pos * CH, 2 * CH)])
    T = pos

    def zi_body(rr, cc):
        for slot in range(2):
            for i in range(CH // 16):
                srcg[slot, rr, pl.ds(i * 16, 16)] = zeros16i
        return cc

    lax.fori_loop(0, GS, zi_body, 0)

    plsc.subcore_barrier()

    def gdesc(j, rows, sem):
        slot = (j // GS) % 2
        return pltpu.make_async_copy(hd_hbm.at[srcg.at[slot, j % GS]],
                                     rows, sem)

    def iload(g):
        slot = g % 2
        pltpu.sync_copy(lsrc_hbm.at[pl.ds(tb + g * GS * CH, GS * CH)],
                        ibufS.at[slot])
        pltpu.sync_copy(ldst_hbm.at[pl.ds(tb + g * GS * CH, GS * CH)],
                        ibufD.at[slot])

        def rs(rr, cc):
            for i in range(CH // 16):
                srcg[slot, rr, pl.ds(i * 16, 16)] = (
                    ibufS[slot, pl.ds(rr * CH + i * 16, 16)])
                dstg[slot, rr, pl.ds(i * 16, 16)] = (
                    ibufD[slot, pl.ds(rr * CH + i * 16, 16)])
            return cc

        lax.fori_loop(0, GS, rs, 0)

    @pl.when(T > 0)
    def _():
        iload(0)
        gdesc(0, rows0, sem0).start()
        gdesc(1, rows1, sem1).start()

        def pb_loop(t, cc):
            j0 = 2 * t
            j1 = j0 + 1

            @pl.when(j0 % GS == GS - 2)
            def _():
                iload((j0 + 2) // GS)

            gdesc(j0, rows0, sem0).wait()
            pltpu.sync_copy(rows0, acc.at[dstg.at[(j0 // GS) % 2, j0 % GS]],
                            add=True)
            gdesc(j0 + 2, rows0, sem0).start()
            gdesc(j1, rows1, sem1).wait()
            pltpu.sync_copy(rows1, acc.at[dstg.at[(j1 // GS) % 2, j1 % GS]],
                            add=True)
            gdesc(j1 + 2, rows1, sem1).start()
            return cc

        lax.fori_loop(0, T // 2, pb_loop, 0)

        @pl.when(T % 2 == 1)
        def _():
            j = T - 1
            gdesc(j, rows0, sem0).wait()
            pltpu.sync_copy(rows0, acc.at[dstg.at[(j // GS) % 2, j % GS]],
                            add=True)
            gdesc(j + 2, rows0, sem0).start()

        @pl.when(T % 2 == 0)
        def _():
            gdesc(T, rows0, sem0).wait()
            gdesc(T + 1, rows1, sem1).wait()

        @pl.when(T % 2 == 1)
        def _():
            gdesc(T, rows1, sem1).wait()
            gdesc(T + 1, rows0, sem0).wait()

    plsc.subcore_barrier()
    pltpu.sync_copy(acc.at[pl.ds(s * RPT, RPT)],
                    out_hbm.at[c, pl.ds(s * RPT, RPT)])


def _sc_scatter(hd, src_w, dst_w):
    out = pl.kernel(
        _sc_scatter_body,
        out_type=(
            jax.ShapeDtypeStruct((NC, ACC_H, D), jnp.float32),
            jax.ShapeDtypeStruct((NC * NS * CAPC * CH,), jnp.int32),
            jax.ShapeDtypeStruct((NC * NS * CAPC * CH,), jnp.int32),
        ),
        mesh=plsc.VectorSubcoreMesh(core_axis_name="c", subcore_axis_name="s"),
        scratch_types=[
            pltpu.VMEM((2, GS, CH), jnp.int32),
            pltpu.VMEM((2, GS, CH), jnp.int32),
            pltpu.VMEM((2, GS * CH), jnp.int32),
            pltpu.VMEM((2, GS * CH), jnp.int32),
            pltpu.VMEM((CH, D), jnp.float32),
            pltpu.VMEM((CH, D), jnp.float32),
            pltpu.VMEM((2 * STG,), jnp.int32),
            pltpu.VMEM((2 * STG,), jnp.int32),
            pltpu.VMEM_SHARED((ACC_H, D), jnp.float32),
            pltpu.SemaphoreType.DMA,
            pltpu.SemaphoreType.DMA,
            pltpu.SemaphoreType.DMA,
            pltpu.SemaphoreType.DMA,
            pltpu.SemaphoreType.DMA,
            pltpu.SemaphoreType.DMA,
        ],
        compiler_params=pltpu.CompilerParams(needs_layout_passes=False),
    )(hd, src_w, dst_w)
    return out[0]


def _tc_dis_body(hists_ref, dis_ref):
    deg = jnp.sum(hists_ref[...], axis=0) + 1.0
    dis_ref[...] = lax.rsqrt(deg)[:, None]


def _tc_dis(hists):
    return pl.pallas_call(
        _tc_dis_body,
        out_shape=jax.ShapeDtypeStruct((N_HIST, 1), jnp.float32),
    )(hists)


def _tc_hd1_body(x_ref, w1_ref, dis_ref, hd1_ref):
    h = jnp.dot(x_ref[...], w1_ref[...], preferred_element_type=jnp.float32,
                precision=HIGH)
    hd1_ref[...] = h * dis_ref[...]


def _tc_hd1(x, w1, dis):
    return pl.pallas_call(
        _tc_hd1_body,
        grid=(N // BLK,),
        in_specs=[
            pl.BlockSpec((BLK, D), lambda i: (i, 0)),
            pl.BlockSpec((D, D), lambda i: (0, 0)),
            pl.BlockSpec((BLK, 1), lambda i: (i, 0)),
        ],
        out_specs=pl.BlockSpec((BLK, D), lambda i: (i, 0)),
        out_shape=jax.ShapeDtypeStruct((N, D), jnp.float32),
    )(x, w1, dis)


def _acc_spec():
    return pl.BlockSpec((1, BLK, D), lambda i: (i // 5, i % 5, 0))


def _tc_layer_body(acc_ref, hd_ref, dis_ref, b_ref, g_ref, be_ref, w_ref,
                   h_ref, hdn_ref):
    dis = dis_ref[...]
    agg = acc_ref[0] + hd_ref[...]
    out = dis * agg + b_ref[0, :]
    bnscale = g_ref[0, :] / jnp.sqrt(1.0 + EPS)
    h = jnp.maximum(out * bnscale + be_ref[0, :], 0.0)
    h_ref[...] = h
    hdn = jnp.dot(h, w_ref[...], preferred_element_type=jnp.float32,
                  precision=HIGH)
    hdn_ref[...] = hdn * dis


def _tc_layer(acc, hd, dis, b, g, be, w):
    return pl.pallas_call(
        _tc_layer_body,
        grid=(N // BLK,),
        in_specs=[
            _acc_spec(),
            pl.BlockSpec((BLK, D), lambda i: (i, 0)),
            pl.BlockSpec((BLK, 1), lambda i: (i, 0)),
            pl.BlockSpec((1, D), lambda i: (0, 0)),
            pl.BlockSpec((1, D), lambda i: (0, 0)),
            pl.BlockSpec((1, D), lambda i: (0, 0)),
            pl.BlockSpec((D, D), lambda i: (0, 0)),
        ],
        out_specs=[
            pl.BlockSpec((BLK, D), lambda i: (i, 0)),
            pl.BlockSpec((BLK, D), lambda i: (i, 0)),
        ],
        out_shape=[
            jax.ShapeDtypeStruct((N, D), jnp.float32),
            jax.ShapeDtypeStruct((N, D), jnp.float32),
        ],
    )(acc, hd, dis, b, g, be, w)


def _tc_final_body(acc_ref, hd_ref, dis_ref, b_ref, g_ref, be_ref,
                   x_ref, h1_ref, batch_ref,
                   p0_ref, p1_ref, p2_ref, pb0_ref, pb1_ref, pb2_ref,
                   score_ref):
    agg = acc_ref[0] + hd_ref[...]
    out = dis_ref[...] * agg + b_ref[0, :]
    bnscale = g_ref[0, :] / jnp.sqrt(1.0 + EPS)
    h2 = jnp.maximum(out * bnscale + be_ref[0, :], 0.0)

    t = jnp.dot(x_ref[...], p0_ref[...], preferred_element_type=jnp.float32,
                precision=HIGH)
    t += jnp.dot(h1_ref[...], p1_ref[...], preferred_element_type=jnp.float32,
                 precision=HIGH)
    t += jnp.dot(h2, p2_ref[...], preferred_element_type=jnp.float32,
                 precision=HIGH)

    b = batch_ref[0, 0, :]
    gio = lax.broadcasted_iota(jnp.int32, (BLK, G), 1)
    onehot = (b[:, None] == gio).astype(jnp.float32)
    contrib = lax.dot_general(onehot, t, (((0,), (0,)), ((), ())),
                              preferred_element_type=jnp.float32,
                              precision=HIGH)

    @pl.when(pl.program_id(0) == 0)
    def _():
        pbs = pb0_ref[0, :] + pb1_ref[0, :] + pb2_ref[0, :]
        score_ref[...] = jnp.broadcast_to(pbs[None, :], (G, D_OUT))

    score_ref[...] += contrib


def _tc_final(acc, hd, dis, b, g, be, x, h1, batch_r, p0, p1, p2,
              pb0, pb1, pb2):
    return pl.pallas_call(
        _tc_final_body,
        grid=(N // BLK,),
        in_specs=[
            _acc_spec(),
            pl.BlockSpec((BLK, D), lambda i: (i, 0)),
            pl.BlockSpec((BLK, 1), lambda i: (i, 0)),
            pl.BlockSpec((1, D), lambda i: (0, 0)),
            pl.BlockSpec((1, D), lambda i: (0, 0)),
            pl.BlockSpec((1, D), lambda i: (0, 0)),
            pl.BlockSpec((BLK, D), lambda i: (i, 0)),
            pl.BlockSpec((BLK, D), lambda i: (i, 0)),
            pl.BlockSpec((1, 1, BLK), lambda i: (i, 0, 0)),
            pl.BlockSpec((D, D_OUT), lambda i: (0, 0)),
            pl.BlockSpec((D, D_OUT), lambda i: (0, 0)),
            pl.BlockSpec((D, D_OUT), lambda i: (0, 0)),
            pl.BlockSpec((1, D_OUT), lambda i: (0, 0)),
            pl.BlockSpec((1, D_OUT), lambda i: (0, 0)),
            pl.BlockSpec((1, D_OUT), lambda i: (0, 0)),
        ],
        out_specs=pl.BlockSpec((G, D_OUT), lambda i: (0, 0)),
        out_shape=jax.ShapeDtypeStruct((G, D_OUT), jnp.float32),
    )(acc, hd, dis, b, g, be, x, h1, batch_r, p0, p1, p2, pb0, pb1, pb2)


def kernel(x, edge_index, batch, W1, b1, g1, be1, W2, b2, g2, be2,
           P0, pb0, P1, pb1, P2, pb2):
    src, dst = edge_index[0], edge_index[1]
    pad = NCHT * CH - EPT
    src_w = jnp.pad(src.reshape(NS, EPT), ((0, 0), (0, pad))).reshape(NS, NCHT, CH)
    dst_w = jnp.pad(dst.reshape(NS, EPT), ((0, 0), (0, pad)),
                    constant_values=N).reshape(NS, NCHT, CH)

    hists = _sc_degree(dst_w)
    dis = _tc_dis(hists)

    hd1 = _tc_hd1(x, W1, dis)
    acc1 = _sc_scatter(hd1, src_w, dst_w)
    h1, hd2 = _tc_layer(acc1, hd1, dis,
                        b1.reshape(1, D), g1.reshape(1, D), be1.reshape(1, D),
                        W2)
    acc2 = _sc_scatter(hd2, src_w, dst_w)

    batch_r = batch.reshape(N // BLK, 1, BLK)
    score = _tc_final(acc2, hd2, dis,
                      b2.reshape(1, D), g2.reshape(1, D), be2.reshape(1, D),
                      x, h1, batch_r,
                      P0, P1, P2,
                      pb0.reshape(1, D_OUT), pb1.reshape(1, D_OUT),
                      pb2.reshape(1, D_OUT))
    return score
